# baseline (speedup 1.0000x reference)
import functools

import jax
import jax.numpy as jnp
from jax import lax
from jax.experimental import pallas as pl
from jax.experimental.pallas import tpu as pltpu
from jax.experimental.pallas import tpu_sc as plsc

_N = 10000
_E = 320000
_D = 128
_H = 128
_FF = 256
_EPS = 1e-5

_TILES = 32
_EPT = _E // _TILES
_CHUNK = 80
_NCH = _EPT // _CHUNK
_DTILES = 10
_DROWS = _N // _DTILES
_DRAIN = 200
_DEGC = 1000

_BLK = 1000
_GRID = _N // _BLK


def _sc_body(want_deg, *refs):
    if want_deg:
        (h_hbm, src_hbm, dst_hbm, z2d_hbm, z1d_hbm, ones_hbm,
         agg_out, deg_out,
         sidx_v, dsts_v, rows0_v, rows1_v, rows2_v, ones_v, dstg_v,
         agg_sh, deg_sh,
         semg0, semg1, semg2, sems0, sems1, sems2,
         semi0, semi1, semi2, semd0, semd1, semd2) = refs
        semd = (semd0, semd1, semd2)
    else:
        (h_hbm, src_hbm, dst_hbm, z2d_hbm,
         agg_out,
         sidx_v, dsts_v, rows0_v, rows1_v, rows2_v,
         agg_sh,
         semg0, semg1, semg2, sems0, sems1, sems2,
         semi0, semi1, semi2) = refs
    rows = (rows0_v, rows1_v, rows2_v)
    semg = (semg0, semg1, semg2)
    sems = (sems0, sems1, sems2)
    semi = (semi0, semi1, semi2)

    c = lax.axis_index("c")
    s = lax.axis_index("s")
    wid = s * 2 + c

    pltpu.sync_copy(dst_hbm.at[wid], dsts_v)

    def idx_start(i, u):
        pltpu.make_async_copy(
            src_hbm.at[pl.ds(wid * _EPT + i * _CHUNK, _CHUNK)],
            sidx_v.at[u], semi[u]).start()

    def idx_wait(i, u):
        pltpu.make_async_copy(
            src_hbm.at[pl.ds(wid * _EPT + i * _CHUNK, _CHUNK)],
            sidx_v.at[u], semi[u]).wait()

    if want_deg:
        pltpu.sync_copy(ones_hbm, ones_v)

    @pl.when(s < _DTILES)
    def _():
        for j in range(_DROWS // _DRAIN):
            pltpu.sync_copy(
                z2d_hbm, agg_sh.at[pl.ds(s * _DROWS + j * _DRAIN, _DRAIN)])
        if want_deg:
            pltpu.sync_copy(z1d_hbm, dstg_v)
            pltpu.sync_copy(dstg_v, deg_sh.at[pl.ds(s * _DEGC, _DEGC)])

    plsc.subcore_barrier()

    def gather_start(u):
        pltpu.make_async_copy(
            h_hbm.at[sidx_v.at[u]], rows[u], semg[u]).start()

    def gather_wait(u):
        pltpu.make_async_copy(
            h_hbm.at[sidx_v.at[u]], rows[u], semg[u]).wait()

    def rowsc_start(i, u):
        pltpu.make_async_copy(
            rows[u], agg_sh.at[dsts_v.at[i]], sems[u]).start(add=True)

    def rowsc_wait(i, u):
        pltpu.make_async_copy(
            rows[u], agg_sh.at[dsts_v.at[i]], sems[u]).wait()

    def degsc_start(i, u):
        pltpu.make_async_copy(
            ones_v, deg_sh.at[dsts_v.at[i]], semd[u]).start(add=True)

    def degsc_wait(i, u):
        pltpu.make_async_copy(
            ones_v, deg_sh.at[dsts_v.at[i]], semd[u]).wait()

    idx_start(0, 0)
    idx_start(1, 1)
    idx_start(2, 2)
    idx_wait(0, 0)
    gather_start(0)

    def chunk(i, u, tail):
        up1 = (u + 1) % 3
        gather_wait(u)
        if want_deg:
            if tail:
                degsc_wait(i - 3, u)
            else:
                @pl.when(i >= 3)
                def _():
                    degsc_wait(i - 3, u)
            degsc_start(i, u)
        rowsc_start(i, u)
        if not tail:
            @pl.when(i + 1 < _NCH)
            def _():
                @pl.when(i >= 2)
                def _():
                    rowsc_wait(i - 2, up1)
                idx_wait(i + 1, up1)
                gather_start(up1)

            @pl.when(i + 3 < _NCH)
            def _():
                idx_start(i + 3, u)
        elif i + 1 < _NCH:
            rowsc_wait(i - 2, up1)
            idx_wait(i + 1, up1)
            gather_start(up1)

    def step(g, carry):
        for u in range(3):
            chunk(g * 3 + u, u, tail=False)
        return carry

    lax.fori_loop(0, _NCH // 3, step, 0)
    chunk(_NCH - 2, 0, tail=True)
    chunk(_NCH - 1, 1, tail=True)
    rowsc_wait(_NCH - 3, 2)
    rowsc_wait(_NCH - 2, 0)
    rowsc_wait(_NCH - 1, 1)
    if want_deg:
        degsc_wait(_NCH - 3, 2)
        degsc_wait(_NCH - 2, 0)
        degsc_wait(_NCH - 1, 1)

    plsc.subcore_barrier()

    @pl.when(s < _DTILES)
    def _():
        pltpu.sync_copy(agg_sh.at[pl.ds(s * _DROWS, _DROWS)],
                        agg_out.at[pl.ds(c * _N + s * _DROWS, _DROWS)])
        if want_deg:
            pltpu.sync_copy(deg_sh.at[pl.ds(s * _DEGC, _DEGC)], dstg_v)
            pltpu.sync_copy(dstg_v, deg_out.at[pl.ds(c * _N + s * _DEGC, _DEGC)])


def _build_sc_agg(want_deg):
    mesh = plsc.VectorSubcoreMesh(core_axis_name="c", subcore_axis_name="s",
                                  num_cores=2, num_subcores=16)
    out_type = [jax.ShapeDtypeStruct((2 * _N, _D), jnp.float32)]
    scratch = [
        pltpu.VMEM((3, _CHUNK), jnp.int32),
        pltpu.VMEM((_NCH, _CHUNK), jnp.int32),
        pltpu.VMEM((_CHUNK, _D), jnp.float32),
        pltpu.VMEM((_CHUNK, _D), jnp.float32),
        pltpu.VMEM((_CHUNK, _D), jnp.float32),
    ]
    if want_deg:
        out_type.append(jax.ShapeDtypeStruct((2 * _N,), jnp.float32))
        scratch += [
            pltpu.VMEM((_CHUNK,), jnp.float32),
            pltpu.VMEM((_DEGC,), jnp.float32),
        ]
    scratch += [
        pltpu.VMEM_SHARED((_N, _D), jnp.float32),
    ]
    if want_deg:
        scratch.append(pltpu.VMEM_SHARED((_N,), jnp.float32))
    scratch += [pltpu.SemaphoreType.DMA] * 9
    if want_deg:
        scratch += [pltpu.SemaphoreType.DMA] * 3

    return pl.kernel(
        functools.partial(_sc_body, want_deg),
        out_type=out_type,
        mesh=mesh,
        scratch_types=scratch,
    )


def _ln(t, g, b):
    m = jnp.mean(t, axis=1, keepdims=True)
    d = t - m
    v = jnp.mean(d * d, axis=1, keepdims=True)
    return d * lax.rsqrt(v + _EPS) * g + b


def _dense_body(x_ref, a0_ref, a1_ref, d0_ref, d1_ref,
                wg_ref, bg_ref, g1_ref, be1_ref, w1_ref, b1_ref, w2_ref,
                b2_ref, g2_ref, be2_ref, z_ref):
    deg_inv = 1.0 / (d0_ref[...] + d1_ref[...] + 1.0)
    s = (a0_ref[...] + a1_ref[...] + x_ref[...]) * deg_inv
    a = jnp.dot(s, wg_ref[...], preferred_element_type=jnp.float32) + bg_ref[...]
    y = _ln(x_ref[...] + a, g1_ref[...], be1_ref[...])
    u = jnp.maximum(
        jnp.dot(y, w1_ref[...], preferred_element_type=jnp.float32)
        + b1_ref[...], 0.0)
    u = jnp.dot(u, w2_ref[...], preferred_element_type=jnp.float32) + b2_ref[...]
    z_ref[...] = _ln(y + u, g2_ref[...], be2_ref[...])


def _tc_dense(x, agg_parts, deg_parts, p):
    row = lambda i: (i, 0)
    full = lambda i: (0, 0)
    in_specs = [
        pl.BlockSpec((_BLK, _H), row),
        pl.BlockSpec((_BLK, _H), row),
        pl.BlockSpec((_BLK, _H), lambda i: (i + _GRID, 0)),
        pl.BlockSpec((_BLK, 1), row),
        pl.BlockSpec((_BLK, 1), lambda i: (i + _GRID, 0)),
        pl.BlockSpec((_H, _H), full),
        pl.BlockSpec((1, _H), full),
        pl.BlockSpec((1, _H), full),
        pl.BlockSpec((1, _H), full),
        pl.BlockSpec((_H, _FF), full),
        pl.BlockSpec((1, _FF), full),
        pl.BlockSpec((_FF, _H), full),
        pl.BlockSpec((1, _H), full),
        pl.BlockSpec((1, _H), full),
        pl.BlockSpec((1, _H), full),
    ]
    args = [x, agg_parts, agg_parts, deg_parts, deg_parts,
            p['Wg'], p['bg'].reshape(1, _H), p['g1'].reshape(1, _H),
            p['be1'].reshape(1, _H), p['W1'], p['b1'].reshape(1, _FF),
            p['W2'], p['b2'].reshape(1, _H), p['g2'].reshape(1, _H),
            p['be2'].reshape(1, _H)]
    return pl.pallas_call(
        _dense_body,
        grid=(_GRID,),
        in_specs=in_specs,
        out_specs=pl.BlockSpec((_BLK, _H), row),
        out_shape=jax.ShapeDtypeStruct((_N, _H), jnp.float32),
    )(*args)


def kernel(x, edge_index, params):
    assert edge_index.shape == (2, _E) and x.shape == (_N, _D)
    src1d = edge_index[0]
    dst3d = edge_index[1].reshape(_TILES, _NCH, _CHUNK)
    z2d = jnp.zeros((_DRAIN, _D), jnp.float32)
    z1d = jnp.zeros((_DEGC,), jnp.float32)
    ones = jnp.ones((_CHUNK,), jnp.float32)

    p0, p1 = params
    aggp, degp = _build_sc_agg(True)(x, src1d, dst3d, z2d, z1d, ones)
    deg2 = degp.reshape(2 * _N, 1)
    z1 = _tc_dense(x, aggp, deg2, p0)
    (aggp2,) = _build_sc_agg(False)(z1, src1d, dst3d, z2d)
    out = _tc_dense(z1, aggp2, deg2, p1)
    return out


# speedup vs baseline: 1.2466x; 1.2466x over previous
import functools

import jax
import jax.numpy as jnp
from jax import lax
from jax.experimental import pallas as pl
from jax.experimental.pallas import tpu as pltpu
from jax.experimental.pallas import tpu_sc as plsc

_N = 10000
_E = 320000
_D = 128
_H = 128
_FF = 256
_EPS = 1e-5

_TILES = 32
_EPT = _E // _TILES
_CHUNK = 80
_NCH = _EPT // _CHUNK
_DTILES = 10
_DROWS = _N // _DTILES
_DRAIN = 200
_DEGC = 1000

_BLK = 2000
_GRID = _N // _BLK


def _sc_body(want_deg, *refs):
    if want_deg:
        (h_hbm, src_hbm, dst_hbm, z2d_hbm, z1d_hbm, ones_hbm,
         agg_out, deg_out,
         srcs_v, dsts_v, rows0_v, rows1_v, ones_v, dstg_v,
         agg_sh, deg_sh, semg0, semg1, semd0, semd1) = refs
        semd = (semd0, semd1)
    else:
        (h_hbm, src_hbm, dst_hbm, z2d_hbm,
         agg_out,
         srcs_v, dsts_v, rows0_v, rows1_v,
         agg_sh, semg0, semg1) = refs
    rows = (rows0_v, rows1_v)
    semg = (semg0, semg1)

    c = lax.axis_index("c")
    s = lax.axis_index("s")
    wid = s * 2 + c

    pltpu.sync_copy(src_hbm.at[pl.ds(wid * _EPT, _EPT)], srcs_v)
    pltpu.sync_copy(dst_hbm.at[wid], dsts_v)

    if want_deg:
        pltpu.sync_copy(ones_hbm, ones_v)

    @pl.when(s < _DTILES)
    def _():
        for j in range(_DROWS // _DRAIN):
            pltpu.sync_copy(
                z2d_hbm, agg_sh.at[pl.ds(s * _DROWS + j * _DRAIN, _DRAIN)])
        if want_deg:
            pltpu.sync_copy(z1d_hbm, dstg_v)
            pltpu.sync_copy(dstg_v, deg_sh.at[pl.ds(s * _DEGC, _DEGC)])

    plsc.subcore_barrier()

    def sidx(i):
        return srcs_v.at[pl.ds(i * _CHUNK, _CHUNK)]

    pltpu.async_copy(h_hbm.at[sidx(0)], rows[0], semg[0])
    pltpu.async_copy(h_hbm.at[sidx(1)], rows[1], semg[1])

    def chunk(i, b, last):
        pltpu.make_async_copy(h_hbm.at[sidx(i)], rows[b], semg[b]).wait()
        if want_deg:
            if not last:
                @pl.when(i >= 2)
                def _():
                    pltpu.make_async_copy(
                        ones_v, deg_sh.at[dsts_v.at[i - 2]], semd[b]).wait()
            pltpu.make_async_copy(
                ones_v, deg_sh.at[dsts_v.at[i]], semd[b]).start(add=True)
        pltpu.sync_copy(rows[b], agg_sh.at[dsts_v.at[i]], add=True)
        if not last:
            @pl.when(i + 2 < _NCH)
            def _():
                pltpu.async_copy(h_hbm.at[sidx(i + 2)], rows[b], semg[b])

    def step(g, carry):
        for b in range(2):
            chunk(g * 2 + b, b, last=False)
        return carry

    lax.fori_loop(0, _NCH // 2, step, 0)
    chunk(_NCH - 1, 0, last=True)
    if want_deg:
        pltpu.make_async_copy(
            ones_v, deg_sh.at[dsts_v.at[_NCH - 3]], semd[0]).wait()
        pltpu.make_async_copy(
            ones_v, deg_sh.at[dsts_v.at[_NCH - 1]], semd[0]).wait()
        pltpu.make_async_copy(
            ones_v, deg_sh.at[dsts_v.at[_NCH - 2]], semd[1]).wait()

    plsc.subcore_barrier()

    @pl.when(s < _DTILES)
    def _():
        pltpu.sync_copy(agg_sh.at[pl.ds(s * _DROWS, _DROWS)],
                        agg_out.at[pl.ds(c * _N + s * _DROWS, _DROWS)])
        if want_deg:
            pltpu.sync_copy(deg_sh.at[pl.ds(s * _DEGC, _DEGC)], dstg_v)
            pltpu.sync_copy(dstg_v, deg_out.at[pl.ds(c * _N + s * _DEGC, _DEGC)])


def _build_sc_agg(want_deg):
    mesh = plsc.VectorSubcoreMesh(core_axis_name="c", subcore_axis_name="s",
                                  num_cores=2, num_subcores=16)
    out_type = [jax.ShapeDtypeStruct((2 * _N, _D), jnp.float32)]
    scratch = [
        pltpu.VMEM((_EPT,), jnp.int32),
        pltpu.VMEM((_NCH, _CHUNK), jnp.int32),
        pltpu.VMEM((_CHUNK, _D), jnp.float32),
        pltpu.VMEM((_CHUNK, _D), jnp.float32),
    ]
    if want_deg:
        out_type.append(jax.ShapeDtypeStruct((2 * _N,), jnp.float32))
        scratch += [
            pltpu.VMEM((_CHUNK,), jnp.float32),
            pltpu.VMEM((_DEGC,), jnp.float32),
        ]
    scratch += [
        pltpu.VMEM_SHARED((_N, _D), jnp.float32),
    ]
    if want_deg:
        scratch.append(pltpu.VMEM_SHARED((_N,), jnp.float32))
    scratch += [pltpu.SemaphoreType.DMA, pltpu.SemaphoreType.DMA]
    if want_deg:
        scratch += [pltpu.SemaphoreType.DMA, pltpu.SemaphoreType.DMA]

    return pl.kernel(
        functools.partial(_sc_body, want_deg),
        out_type=out_type,
        mesh=mesh,
        scratch_types=scratch,
    )


def _ln(t, g, b):
    m = jnp.mean(t, axis=1, keepdims=True)
    d = t - m
    v = jnp.mean(d * d, axis=1, keepdims=True)
    return d * lax.rsqrt(v + _EPS) * g + b


def _dense_body(x_ref, a0_ref, a1_ref, d0_ref, d1_ref,
                wg_ref, bg_ref, g1_ref, be1_ref, w1_ref, b1_ref, w2_ref,
                b2_ref, g2_ref, be2_ref, z_ref):
    deg_inv = 1.0 / (d0_ref[...] + d1_ref[...] + 1.0)
    s = (a0_ref[...] + a1_ref[...] + x_ref[...]) * deg_inv
    a = jnp.dot(s, wg_ref[...], preferred_element_type=jnp.float32) + bg_ref[...]
    y = _ln(x_ref[...] + a, g1_ref[...], be1_ref[...])
    u = jnp.maximum(
        jnp.dot(y, w1_ref[...], preferred_element_type=jnp.float32)
        + b1_ref[...], 0.0)
    u = jnp.dot(u, w2_ref[...], preferred_element_type=jnp.float32) + b2_ref[...]
    z_ref[...] = _ln(y + u, g2_ref[...], be2_ref[...])


def _tc_dense(x, agg_parts, deg_parts, p):
    row = lambda i: (i, 0)
    full = lambda i: (0, 0)
    in_specs = [
        pl.BlockSpec((_BLK, _H), row),
        pl.BlockSpec((_BLK, _H), row),
        pl.BlockSpec((_BLK, _H), lambda i: (i + _GRID, 0)),
        pl.BlockSpec((_BLK, 1), row),
        pl.BlockSpec((_BLK, 1), lambda i: (i + _GRID, 0)),
        pl.BlockSpec((_H, _H), full),
        pl.BlockSpec((1, _H), full),
        pl.BlockSpec((1, _H), full),
        pl.BlockSpec((1, _H), full),
        pl.BlockSpec((_H, _FF), full),
        pl.BlockSpec((1, _FF), full),
        pl.BlockSpec((_FF, _H), full),
        pl.BlockSpec((1, _H), full),
        pl.BlockSpec((1, _H), full),
        pl.BlockSpec((1, _H), full),
    ]
    args = [x, agg_parts, agg_parts, deg_parts, deg_parts,
            p['Wg'], p['bg'].reshape(1, _H), p['g1'].reshape(1, _H),
            p['be1'].reshape(1, _H), p['W1'], p['b1'].reshape(1, _FF),
            p['W2'], p['b2'].reshape(1, _H), p['g2'].reshape(1, _H),
            p['be2'].reshape(1, _H)]
    return pl.pallas_call(
        _dense_body,
        grid=(_GRID,),
        in_specs=in_specs,
        out_specs=pl.BlockSpec((_BLK, _H), row),
        out_shape=jax.ShapeDtypeStruct((_N, _H), jnp.float32),
    )(*args)


def kernel(x, edge_index, params):
    assert edge_index.shape == (2, _E) and x.shape == (_N, _D)
    src1d = edge_index[0]
    dst3d = edge_index[1].reshape(_TILES, _NCH, _CHUNK)
    z2d = jnp.zeros((_DRAIN, _D), jnp.float32)
    z1d = jnp.zeros((_DEGC,), jnp.float32)
    ones = jnp.ones((_CHUNK,), jnp.float32)

    p0, p1 = params
    aggp, degp = _build_sc_agg(True)(x, src1d, dst3d, z2d, z1d, ones)
    deg2 = degp.reshape(2 * _N, 1)
    z1 = _tc_dense(x, aggp, deg2, p0)
    (aggp2,) = _build_sc_agg(False)(z1, src1d, dst3d, z2d)
    out = _tc_dense(z1, aggp2, deg2, p1)
    return out
